# Initial kernel scaffold; baseline (speedup 1.0000x reference)
import jax
import jax.numpy as jnp
from jax.experimental import pallas as pl


def kernel(z_e_x, codebook):
    raise NotImplementedError("write your pallas kernel here")


# speedup vs baseline: 1.0134x; 1.0134x over previous
import jax
import jax.numpy as jnp
from jax.experimental import pallas as pl


def _vq_kernel(x_ref, cb_ref, out_ref):
    x = x_ref[...]
    cb = cb_ref[...]
    xlo = x[:, :128]
    xhi = x[:, 128:]
    clo = cb[:, :128]
    chi = cb[:, 128:]
    in_sqr = jnp.sum(xlo * xlo + xhi * xhi, axis=1, keepdims=True)
    cb_sqr = jnp.sum(clo * clo + chi * chi, axis=1)
    mm = jax.lax.dot_general(
        x.astype(jnp.bfloat16), cb.astype(jnp.bfloat16),
        (((1,), (1,)), ((), ())),
        preferred_element_type=jnp.float32)
    dist = in_sqr + cb_sqr[None, :] - 2.0 * mm
    minv = jnp.min(dist, axis=1, keepdims=True)
    iota = jax.lax.broadcasted_iota(jnp.int32, dist.shape, 1)
    cand = jnp.where(dist == minv, iota, jnp.int32(dist.shape[1]))
    out_ref[...] = jnp.min(cand, axis=1).reshape(1, 1, -1)


def kernel(z_e_x, codebook):
    B, D, H, W = z_e_x.shape
    K = codebook.shape[0]
    flat = jnp.transpose(z_e_x, (0, 2, 3, 1)).reshape(-1, D)
    N = flat.shape[0]
    BN = 2048
    NB = N // BN
    out = pl.pallas_call(
        _vq_kernel,
        grid=(NB,),
        in_specs=[
            pl.BlockSpec((BN, D), lambda i: (i, 0)),
            pl.BlockSpec((K, D), lambda i: (0, 0)),
        ],
        out_specs=pl.BlockSpec((1, 1, BN), lambda i: (i, 0, 0)),
        out_shape=jax.ShapeDtypeStruct((NB, 1, BN), jnp.int32),
    )(flat, codebook)
    return out.reshape(B, H, W)
